# Initial kernel scaffold; baseline (speedup 1.0000x reference)
import jax
import jax.numpy as jnp
from jax.experimental import pallas as pl


def kernel(x, W1, b1, W2, b2):
    raise NotImplementedError("write your pallas kernel here")


# speedup vs baseline: 2.5957x; 2.5957x over previous
import functools

import jax
import jax.numpy as jnp
from jax.experimental import pallas as pl

TOKENS = 8192
D_MODEL = 2048
HIDDEN = 256
N_EXPERTS = 16
TILE = 512


def _gate_body(x_ref, w1_ref, b1_ref, w2_ref, b2_ref, out_ref):
    h = jnp.maximum(
        jnp.dot(x_ref[...], w1_ref[...], preferred_element_type=jnp.float32)
        + b1_ref[...],
        0.0,
    )
    logits = (
        jnp.dot(h, w2_ref[...], preferred_element_type=jnp.float32) + b2_ref[...]
    )
    eidx = jax.lax.broadcasted_iota(jnp.int32, logits.shape, 1)
    m1 = jnp.max(logits, axis=-1, keepdims=True)
    a1 = jnp.min(
        jnp.where(logits == m1, eidx, N_EXPERTS), axis=-1, keepdims=True
    )
    masked = jnp.where(eidx == a1, -jnp.inf, logits)
    m2 = jnp.max(masked, axis=-1, keepdims=True)
    a2 = jnp.min(
        jnp.where(masked == m2, eidx, N_EXPERTS), axis=-1, keepdims=True
    )
    e2 = jnp.exp(m2 - m1)
    g1 = 1.0 / (1.0 + e2)
    g2 = e2 * g1
    out_ref[...] = jnp.where(eidx == a1, g1, jnp.where(eidx == a2, g2, 0.0))


@functools.partial(jax.jit, static_argnames=())
def kernel(x, W1, b1, W2, b2):
    grid = (TOKENS // TILE,)
    return pl.pallas_call(
        _gate_body,
        grid=grid,
        in_specs=[
            pl.BlockSpec((TILE, D_MODEL), lambda i: (i, 0)),
            pl.BlockSpec((D_MODEL, HIDDEN), lambda i: (0, 0)),
            pl.BlockSpec((1, HIDDEN), lambda i: (0, 0)),
            pl.BlockSpec((HIDDEN, N_EXPERTS), lambda i: (0, 0)),
            pl.BlockSpec((1, N_EXPERTS), lambda i: (0, 0)),
        ],
        out_specs=pl.BlockSpec((TILE, N_EXPERTS), lambda i: (i, 0)),
        out_shape=jax.ShapeDtypeStruct((TOKENS, N_EXPERTS), jnp.float32),
    )(x, W1, b1.reshape(1, HIDDEN), W2, b2.reshape(1, N_EXPERTS))


# speedup vs baseline: 3.0754x; 1.1848x over previous
import functools

import jax
import jax.numpy as jnp
from jax.experimental import pallas as pl

TOKENS = 8192
D_MODEL = 2048
HIDDEN = 256
N_EXPERTS = 16
TILE = 1024


def _gate_body(x_ref, w1_ref, b1_ref, w2_ref, b2_ref, out_ref):
    h = jnp.maximum(
        jnp.dot(x_ref[...], w1_ref[...], preferred_element_type=jnp.float32)
        + b1_ref[...],
        0.0,
    )
    logits = (
        jnp.dot(h, w2_ref[...], preferred_element_type=jnp.float32) + b2_ref[...]
    )
    eidx = jax.lax.broadcasted_iota(jnp.int32, logits.shape, 1)
    m1 = jnp.max(logits, axis=-1, keepdims=True)
    a1 = jnp.min(
        jnp.where(logits == m1, eidx, N_EXPERTS), axis=-1, keepdims=True
    )
    masked = jnp.where(eidx == a1, -jnp.inf, logits)
    m2 = jnp.max(masked, axis=-1, keepdims=True)
    a2 = jnp.min(
        jnp.where(masked == m2, eidx, N_EXPERTS), axis=-1, keepdims=True
    )
    e2 = jnp.exp(m2 - m1)
    g1 = 1.0 / (1.0 + e2)
    g2 = e2 * g1
    out_ref[...] = jnp.where(eidx == a1, g1, jnp.where(eidx == a2, g2, 0.0))


@functools.partial(jax.jit, static_argnames=())
def kernel(x, W1, b1, W2, b2):
    grid = (TOKENS // TILE,)
    return pl.pallas_call(
        _gate_body,
        grid=grid,
        in_specs=[
            pl.BlockSpec((TILE, D_MODEL), lambda i: (i, 0)),
            pl.BlockSpec((D_MODEL, HIDDEN), lambda i: (0, 0)),
            pl.BlockSpec((1, HIDDEN), lambda i: (0, 0)),
            pl.BlockSpec((HIDDEN, N_EXPERTS), lambda i: (0, 0)),
            pl.BlockSpec((1, N_EXPERTS), lambda i: (0, 0)),
        ],
        out_specs=pl.BlockSpec((TILE, N_EXPERTS), lambda i: (i, 0)),
        out_shape=jax.ShapeDtypeStruct((TOKENS, N_EXPERTS), jnp.float32),
    )(x, W1, b1.reshape(1, HIDDEN), W2, b2.reshape(1, N_EXPERTS))


# speedup vs baseline: 3.1517x; 1.0248x over previous
import functools

import jax
import jax.numpy as jnp
from jax.experimental import pallas as pl

TOKENS = 8192
D_MODEL = 2048
HIDDEN = 256
N_EXPERTS = 16
TILE = 2048


def _gate_body(x_ref, w1_ref, b1_ref, w2_ref, b2_ref, out_ref):
    h = jnp.maximum(
        jnp.dot(x_ref[...], w1_ref[...], preferred_element_type=jnp.float32)
        + b1_ref[...],
        0.0,
    )
    logits = (
        jnp.dot(h, w2_ref[...], preferred_element_type=jnp.float32) + b2_ref[...]
    )
    eidx = jax.lax.broadcasted_iota(jnp.int32, logits.shape, 1)
    m1 = jnp.max(logits, axis=-1, keepdims=True)
    a1 = jnp.min(
        jnp.where(logits == m1, eidx, N_EXPERTS), axis=-1, keepdims=True
    )
    masked = jnp.where(eidx == a1, -jnp.inf, logits)
    m2 = jnp.max(masked, axis=-1, keepdims=True)
    a2 = jnp.min(
        jnp.where(masked == m2, eidx, N_EXPERTS), axis=-1, keepdims=True
    )
    e2 = jnp.exp(m2 - m1)
    g1 = 1.0 / (1.0 + e2)
    g2 = e2 * g1
    out_ref[...] = jnp.where(eidx == a1, g1, jnp.where(eidx == a2, g2, 0.0))


@functools.partial(jax.jit, static_argnames=())
def kernel(x, W1, b1, W2, b2):
    grid = (TOKENS // TILE,)
    return pl.pallas_call(
        _gate_body,
        grid=grid,
        in_specs=[
            pl.BlockSpec((TILE, D_MODEL), lambda i: (i, 0)),
            pl.BlockSpec((D_MODEL, HIDDEN), lambda i: (0, 0)),
            pl.BlockSpec((1, HIDDEN), lambda i: (0, 0)),
            pl.BlockSpec((HIDDEN, N_EXPERTS), lambda i: (0, 0)),
            pl.BlockSpec((1, N_EXPERTS), lambda i: (0, 0)),
        ],
        out_specs=pl.BlockSpec((TILE, N_EXPERTS), lambda i: (i, 0)),
        out_shape=jax.ShapeDtypeStruct((TOKENS, N_EXPERTS), jnp.float32),
    )(x, W1, b1.reshape(1, HIDDEN), W2, b2.reshape(1, N_EXPERTS))
